# baseline (speedup 1.0000x reference)
import functools

import jax
import jax.numpy as jnp
from jax import lax
from jax.experimental import pallas as pl
from jax.experimental.pallas import tpu as pltpu
from jax.experimental.pallas import tpu_sc as plsc

N = 10000
D = 128
OUT = 32
E = 320000
NC = 2
NS = 16
NW = NC * NS
EPT = E // NW
K = 80
CH = EPT // K
SB = 5
NSB = CH // SB
NPAD = 10240
RPS = NPAD // NS

_f32 = jnp.float32


def _tc_front(x_ref, win_ref, bin_ref, lin1_ref, atts_ref, attd_ref,
              feats_ref, h_ref, asrc_ref, adst_ref, m_ref):
    x = x_ref[...]
    f = jnp.dot(x, win_ref[...], preferred_element_type=_f32) + bin_ref[...]
    f = jnp.where(f >= 0, f, 0.01 * f)
    feats_ref[...] = f
    h = jnp.dot(f, lin1_ref[...], preferred_element_type=_f32)
    h_ref[...] = h
    a_s = jnp.sum(h * atts_ref[...], axis=1)
    a_d = jnp.sum(h * attd_ref[...], axis=1)
    asrc_ref[0, :] = a_s
    adst_ref[0, :] = a_d
    m = jnp.max(a_s) + jnp.max(a_d)
    m_ref[...] = jnp.full((1, 16), m, _f32)


_sc_mesh = plsc.VectorSubcoreMesh(core_axis_name="c", subcore_axis_name="s")


@functools.partial(
    pl.kernel,
    out_type=(
        jax.ShapeDtypeStruct((NW, NSB, SB, K), _f32),
        jax.ShapeDtypeStruct((NW, N), _f32),
    ),
    mesh=_sc_mesh,
    compiler_params=pltpu.CompilerParams(needs_layout_passes=False),
    scratch_types=[
        pltpu.VMEM((CH, K), jnp.int32),
        pltpu.VMEM((CH, K), jnp.int32),
        pltpu.VMEM((CH, K), _f32),
        pltpu.VMEM((N,), _f32),
        pltpu.VMEM((N,), _f32),
        pltpu.VMEM((N,), _f32),
        pltpu.VMEM((16,), _f32),
        pltpu.SemaphoreType.DMA,
    ],
)
def _sc_scalar(src_hbm, dst_hbm, asrc_hbm, adst_hbm, m_hbm,
               ex_hbm, den_hbm,
               src_v, dst_v, ex_v, asrc_v, adst_v, den_v, m_v, sem):
    cid = lax.axis_index("c")
    sid = lax.axis_index("s")
    wid = sid * NC + cid

    @pl.loop(0, NSB)
    def _stage(sc):
        pltpu.async_copy(src_hbm.at[wid, sc],
                         src_v.at[pl.ds(sc * SB, SB)], sem)
        pltpu.async_copy(dst_hbm.at[wid, sc],
                         dst_v.at[pl.ds(sc * SB, SB)], sem)

    pltpu.async_copy(asrc_hbm.at[0], asrc_v, sem)
    pltpu.async_copy(adst_hbm.at[0], adst_v, sem)
    pltpu.async_copy(m_hbm.at[0], m_v, sem)

    @pl.loop(0, N // 16)
    def _zero_den(i):
        den_v[pl.ds(16 * i, 16)] = jnp.zeros((16,), _f32)

    @pl.loop(0, NSB)
    def _stage_wait(sc):
        pltpu.make_async_copy(src_hbm.at[wid, sc],
                              src_v.at[pl.ds(sc * SB, SB)], sem).wait()
        pltpu.make_async_copy(dst_hbm.at[wid, sc],
                              dst_v.at[pl.ds(sc * SB, SB)], sem).wait()

    pltpu.make_async_copy(asrc_hbm.at[0], asrc_v, sem).wait()
    pltpu.make_async_copy(adst_hbm.at[0], adst_v, sem).wait()
    pltpu.make_async_copy(m_hbm.at[0], m_v, sem).wait()
    mvec = m_v[...]

    @pl.loop(0, CH, unroll=4)
    def _scalar(ci):
        for j in range(K // 16):
            sl = pl.ds(16 * j, 16)
            si = src_v[ci, sl]
            di = dst_v[ci, sl]
            a = (plsc.load_gather(asrc_v, [si])
                 + plsc.load_gather(adst_v, [di]))
            a = jnp.where(a >= 0, a, 0.2 * a)
            exv = jnp.exp(a - mvec)
            ex_v[ci, sl] = exv
            plsc.addupdate_scatter(den_v, [di], exv)

    @pl.loop(0, NSB)
    def _unstage(sc):
        pltpu.sync_copy(ex_v.at[pl.ds(sc * SB, SB)], ex_hbm.at[wid, sc])

    pltpu.sync_copy(den_v, den_hbm.at[wid])


@functools.partial(
    pl.kernel,
    out_type=jax.ShapeDtypeStruct((NC, NPAD, D), _f32),
    mesh=_sc_mesh,
    compiler_params=pltpu.CompilerParams(needs_layout_passes=False),
    scratch_types=[
        pltpu.VMEM((2 * SB, K), jnp.int32),
        pltpu.VMEM((2 * SB, K), jnp.int32),
        pltpu.VMEM((2 * SB, K), _f32),
        pltpu.VMEM((4 * K, D), _f32),
        pltpu.VMEM_SHARED((NPAD, D), _f32),
        pltpu.SemaphoreType.DMA,
        pltpu.SemaphoreType.DMA,
        pltpu.SemaphoreType.DMA,
    ],
)
def _sc_prop(src_hbm, dst_hbm, ex_hbm, table_hbm,
             part_hbm,
             src_v, dst_v, ex_v, rows_v, acc_sh, gsem, ssem, psem):
    cid = lax.axis_index("c")
    sid = lax.axis_index("s")
    wid = sid * NC + cid

    @pl.loop(0, 2 * K)
    def _zero_rows(i):
        for j in range(D // 16):
            rows_v[i, pl.ds(16 * j, 16)] = jnp.zeros((16,), _f32)

    for k in range(RPS // (2 * K)):
        pltpu.sync_copy(rows_v.at[pl.ds(0, 2 * K)],
                        acc_sh.at[pl.ds(sid * RPS + k * 2 * K, 2 * K)])

    plsc.subcore_barrier()

    def _gather_desc(irow, b):
        return pltpu.make_async_copy(table_hbm.at[src_v.at[irow]],
                                     rows_v.at[pl.ds(b * K, K)], gsem)

    def _scatter_fire(irow, b):
        pltpu.async_copy(rows_v.at[pl.ds(b * K, K)],
                         acc_sh.at[dst_v.at[irow]], ssem, add=True)

    def _scatter_drain(irow, b):
        pltpu.make_async_copy(rows_v.at[pl.ds(b * K, K)],
                              acc_sh.at[dst_v.at[irow]], ssem).wait()

    pltpu.sync_copy(src_hbm.at[wid, 0], src_v.at[pl.ds(0, SB)])
    pltpu.sync_copy(dst_hbm.at[wid, 0], dst_v.at[pl.ds(0, SB)])
    pltpu.sync_copy(ex_hbm.at[wid, 0], ex_v.at[pl.ds(0, SB)])
    pltpu.async_copy(table_hbm.at[src_v.at[0]],
                     rows_v.at[pl.ds(0, K)], gsem)
    pltpu.async_copy(table_hbm.at[src_v.at[1]],
                     rows_v.at[pl.ds(K, K)], gsem)
    pltpu.async_copy(table_hbm.at[src_v.at[2]],
                     rows_v.at[pl.ds(2 * K, K)], gsem)

    @pl.loop(0, CH)
    def _iter(ci):
        s = ci // SB
        pos = ci % SB
        b = ci % 4
        half = (s % 2) * SB
        irow = half + pos

        _gather_desc(irow, b).wait()

        @pl.when(ci >= 1)
        def _drain_scatter():
            _scatter_drain(irow, (ci + 3) % 4)

        @pl.when(jnp.logical_and(pos == 1, s + 1 < NSB))
        def _prefetch():
            oh = SB - half
            pltpu.async_copy(src_hbm.at[wid, s + 1],
                             src_v.at[pl.ds(oh, SB)], psem)
            pltpu.async_copy(dst_hbm.at[wid, s + 1],
                             dst_v.at[pl.ds(oh, SB)], psem)
            pltpu.async_copy(ex_hbm.at[wid, s + 1],
                             ex_v.at[pl.ds(oh, SB)], psem)

        @pl.when(jnp.logical_and(pos == 2, s + 1 < NSB))
        def _wait_prefetch():
            oh = SB - half
            pltpu.make_async_copy(src_hbm.at[wid, s + 1],
                                  src_v.at[pl.ds(oh, SB)], psem).wait()
            pltpu.make_async_copy(dst_hbm.at[wid, s + 1],
                                  dst_v.at[pl.ds(oh, SB)], psem).wait()
            pltpu.make_async_copy(ex_hbm.at[wid, s + 1],
                                  ex_v.at[pl.ds(oh, SB)], psem).wait()

        @pl.when(ci + 3 < CH)
        def _next():
            nci = ci + 3
            npos = nci % SB
            nhalf = ((nci // SB) % 2) * SB
            pltpu.async_copy(table_hbm.at[src_v.at[nhalf + npos]],
                             rows_v.at[pl.ds(((ci + 3) % 4) * K, K)], gsem)

        rb = b * K
        for g in range(K // 16):
            exvec = ex_v[irow, pl.ds(16 * g, 16)]
            for el in range(16):
                w = jnp.full((16,), exvec[el], _f32)
                for j in range(D // 16):
                    sl = pl.ds(16 * j, 16)
                    r = rb + 16 * g + el
                    rows_v[r, sl] = rows_v[r, sl] * w

        _scatter_fire(irow, b)

    _scatter_drain(SB - 1, (CH - 1) % 4)

    plsc.subcore_barrier()
    pltpu.sync_copy(acc_sh.at[pl.ds(sid * RPS, RPS)],
                    part_hbm.at[cid, pl.ds(sid * RPS, RPS)])


def _elu(v):
    return jnp.where(v > 0, v, jnp.exp(jnp.minimum(v, 0.0)) - 1.0)


def _tc_mid(part_ref, denp_ref, lin2_ref, h2_ref, h3in_ref, dinv_ref):
    dcol = lax.dot_general(denp_ref[...], jnp.ones((NW, 1), _f32),
                           (((0,), (0,)), ((), ())),
                           preferred_element_type=_f32)
    dinv = 1.0 / (dcol + 1e-16)
    dinv_ref[...] = dinv
    agg = part_ref[0, :N, :] + part_ref[1, :N, :]
    h1 = _elu(agg * dinv)
    h2 = jnp.dot(h1, lin2_ref[...], preferred_element_type=_f32)
    h2_ref[...] = h2
    h3in_ref[...] = lax.dot_general(h2, lin2_ref[...],
                                    (((1,), (1,)), ((), ())),
                                    preferred_element_type=_f32)


def _tc_back(part_ref, dinv_ref, lin1_ref, h4_ref):
    agg = part_ref[0, :N, :] + part_ref[1, :N, :]
    h3 = _elu(agg * dinv_ref[...])
    h4_ref[...] = lax.dot_general(h3, lin1_ref[...],
                                  (((1,), (1,)), ((), ())),
                                  preferred_element_type=_f32)


def kernel(x, edge_index, W_in, b_in, lin1, att_src1, att_dst1, lin2):
    src3 = edge_index[0].reshape(NW, NSB, SB, K)
    dst3 = edge_index[1].reshape(NW, NSB, SB, K)

    feats, h, asrc, adst, m = pl.pallas_call(
        _tc_front,
        out_shape=(
            jax.ShapeDtypeStruct((N, D), _f32),
            jax.ShapeDtypeStruct((N, D), _f32),
            jax.ShapeDtypeStruct((1, N), _f32),
            jax.ShapeDtypeStruct((1, N), _f32),
            jax.ShapeDtypeStruct((1, 16), _f32),
        ),
    )(x, W_in, b_in.reshape(1, D), lin1,
      att_src1.reshape(1, D), att_dst1.reshape(1, D))

    ex, denp = _sc_scalar(src3, dst3, asrc, adst, m)

    part1 = _sc_prop(src3, dst3, ex, h)

    h2, h3in, dinv = pl.pallas_call(
        _tc_mid,
        out_shape=(
            jax.ShapeDtypeStruct((N, OUT), _f32),
            jax.ShapeDtypeStruct((N, D), _f32),
            jax.ShapeDtypeStruct((N, 1), _f32),
        ),
    )(part1, denp, lin2)

    part3 = _sc_prop(src3, dst3, ex, h3in)

    h4 = pl.pallas_call(
        _tc_back,
        out_shape=jax.ShapeDtypeStruct((N, D), _f32),
    )(part3, dinv, lin1)

    return (h2, h4, feats)


# speedup vs baseline: 1.0020x; 1.0020x over previous
import functools

import jax
import jax.numpy as jnp
from jax import lax
from jax.experimental import pallas as pl
from jax.experimental.pallas import tpu as pltpu
from jax.experimental.pallas import tpu_sc as plsc

N = 10000
D = 128
OUT = 32
E = 320000
NC = 2
NS = 16
NW = NC * NS
EPT = E // NW
K = 80
CH = EPT // K
SB = 5
NSB = CH // SB
NPAD = 10240
RPS = NPAD // NS

_f32 = jnp.float32


def _tc_front(x_ref, win_ref, bin_ref, lin1_ref, atts_ref, attd_ref,
              feats_ref, h_ref, asrc_ref, adst_ref, m_ref):
    x = x_ref[...]
    f = jnp.dot(x, win_ref[...], preferred_element_type=_f32) + bin_ref[...]
    f = jnp.where(f >= 0, f, 0.01 * f)
    feats_ref[...] = f
    h = jnp.dot(f, lin1_ref[...], preferred_element_type=_f32)
    h_ref[...] = h
    a_s = jnp.sum(h * atts_ref[...], axis=1)
    a_d = jnp.sum(h * attd_ref[...], axis=1)
    asrc_ref[0, :] = a_s
    adst_ref[0, :] = a_d
    m = jnp.max(a_s) + jnp.max(a_d)
    m_ref[...] = jnp.full((1, 16), m, _f32)


_sc_mesh = plsc.VectorSubcoreMesh(core_axis_name="c", subcore_axis_name="s")


@functools.partial(
    pl.kernel,
    out_type=(
        jax.ShapeDtypeStruct((NW, NSB, SB, K), _f32),
        jax.ShapeDtypeStruct((NW, N), _f32),
    ),
    mesh=_sc_mesh,
    compiler_params=pltpu.CompilerParams(needs_layout_passes=False),
    scratch_types=[
        pltpu.VMEM((CH, K), jnp.int32),
        pltpu.VMEM((CH, K), jnp.int32),
        pltpu.VMEM((CH, K), _f32),
        pltpu.VMEM((N,), _f32),
        pltpu.VMEM((N,), _f32),
        pltpu.VMEM((N,), _f32),
        pltpu.VMEM((16,), _f32),
        pltpu.SemaphoreType.DMA,
    ],
)
def _sc_scalar(src_hbm, dst_hbm, asrc_hbm, adst_hbm, m_hbm,
               ex_hbm, den_hbm,
               src_v, dst_v, ex_v, asrc_v, adst_v, den_v, m_v, sem):
    cid = lax.axis_index("c")
    sid = lax.axis_index("s")
    wid = sid * NC + cid

    @pl.loop(0, NSB)
    def _stage(sc):
        pltpu.async_copy(src_hbm.at[wid, sc],
                         src_v.at[pl.ds(sc * SB, SB)], sem)
        pltpu.async_copy(dst_hbm.at[wid, sc],
                         dst_v.at[pl.ds(sc * SB, SB)], sem)

    pltpu.async_copy(asrc_hbm.at[0], asrc_v, sem)
    pltpu.async_copy(adst_hbm.at[0], adst_v, sem)
    pltpu.async_copy(m_hbm.at[0], m_v, sem)

    @pl.loop(0, N // 16)
    def _zero_den(i):
        den_v[pl.ds(16 * i, 16)] = jnp.zeros((16,), _f32)

    @pl.loop(0, NSB)
    def _stage_wait(sc):
        pltpu.make_async_copy(src_hbm.at[wid, sc],
                              src_v.at[pl.ds(sc * SB, SB)], sem).wait()
        pltpu.make_async_copy(dst_hbm.at[wid, sc],
                              dst_v.at[pl.ds(sc * SB, SB)], sem).wait()

    pltpu.make_async_copy(asrc_hbm.at[0], asrc_v, sem).wait()
    pltpu.make_async_copy(adst_hbm.at[0], adst_v, sem).wait()
    pltpu.make_async_copy(m_hbm.at[0], m_v, sem).wait()
    mvec = m_v[...]

    @pl.loop(0, CH, unroll=2)
    def _scalar(ci):
        for j in range(K // 16):
            sl = pl.ds(16 * j, 16)
            si = src_v[ci, sl]
            di = dst_v[ci, sl]
            a = (plsc.load_gather(asrc_v, [si])
                 + plsc.load_gather(adst_v, [di]))
            a = jnp.where(a >= 0, a, 0.2 * a)
            exv = jnp.exp(a - mvec)
            ex_v[ci, sl] = exv
            plsc.addupdate_scatter(den_v, [di], exv)

    @pl.loop(0, NSB)
    def _unstage(sc):
        pltpu.sync_copy(ex_v.at[pl.ds(sc * SB, SB)], ex_hbm.at[wid, sc])

    pltpu.sync_copy(den_v, den_hbm.at[wid])


@functools.partial(
    pl.kernel,
    out_type=jax.ShapeDtypeStruct((NC, NPAD, D), _f32),
    mesh=_sc_mesh,
    compiler_params=pltpu.CompilerParams(needs_layout_passes=False),
    scratch_types=[
        pltpu.VMEM((2 * SB, K), jnp.int32),
        pltpu.VMEM((2 * SB, K), jnp.int32),
        pltpu.VMEM((2 * SB, K), _f32),
        pltpu.VMEM((4 * K, D), _f32),
        pltpu.VMEM_SHARED((NPAD, D), _f32),
        pltpu.SemaphoreType.DMA,
        pltpu.SemaphoreType.DMA,
        pltpu.SemaphoreType.DMA,
    ],
)
def _sc_prop(src_hbm, dst_hbm, ex_hbm, table_hbm,
             part_hbm,
             src_v, dst_v, ex_v, rows_v, acc_sh, gsem, ssem, psem):
    cid = lax.axis_index("c")
    sid = lax.axis_index("s")
    wid = sid * NC + cid

    @pl.loop(0, 2 * K)
    def _zero_rows(i):
        for j in range(D // 16):
            rows_v[i, pl.ds(16 * j, 16)] = jnp.zeros((16,), _f32)

    for k in range(RPS // (2 * K)):
        pltpu.sync_copy(rows_v.at[pl.ds(0, 2 * K)],
                        acc_sh.at[pl.ds(sid * RPS + k * 2 * K, 2 * K)])

    plsc.subcore_barrier()

    def _gather_desc(irow, b):
        return pltpu.make_async_copy(table_hbm.at[src_v.at[irow]],
                                     rows_v.at[pl.ds(b * K, K)], gsem)

    def _scatter_fire(irow, b):
        pltpu.async_copy(rows_v.at[pl.ds(b * K, K)],
                         acc_sh.at[dst_v.at[irow]], ssem, add=True)

    def _scatter_drain(irow, b):
        pltpu.make_async_copy(rows_v.at[pl.ds(b * K, K)],
                              acc_sh.at[dst_v.at[irow]], ssem).wait()

    pltpu.sync_copy(src_hbm.at[wid, 0], src_v.at[pl.ds(0, SB)])
    pltpu.sync_copy(dst_hbm.at[wid, 0], dst_v.at[pl.ds(0, SB)])
    pltpu.sync_copy(ex_hbm.at[wid, 0], ex_v.at[pl.ds(0, SB)])
    pltpu.async_copy(table_hbm.at[src_v.at[0]],
                     rows_v.at[pl.ds(0, K)], gsem)
    pltpu.async_copy(table_hbm.at[src_v.at[1]],
                     rows_v.at[pl.ds(K, K)], gsem)
    pltpu.async_copy(table_hbm.at[src_v.at[2]],
                     rows_v.at[pl.ds(2 * K, K)], gsem)

    @pl.loop(0, CH)
    def _iter(ci):
        s = ci // SB
        pos = ci % SB
        b = ci % 4
        half = (s % 2) * SB
        irow = half + pos

        _gather_desc(irow, b).wait()

        @pl.when(ci >= 1)
        def _drain_scatter():
            _scatter_drain(irow, (ci + 3) % 4)

        @pl.when(jnp.logical_and(pos == 1, s + 1 < NSB))
        def _prefetch():
            oh = SB - half
            pltpu.async_copy(src_hbm.at[wid, s + 1],
                             src_v.at[pl.ds(oh, SB)], psem)
            pltpu.async_copy(dst_hbm.at[wid, s + 1],
                             dst_v.at[pl.ds(oh, SB)], psem)
            pltpu.async_copy(ex_hbm.at[wid, s + 1],
                             ex_v.at[pl.ds(oh, SB)], psem)

        @pl.when(jnp.logical_and(pos == 2, s + 1 < NSB))
        def _wait_prefetch():
            oh = SB - half
            pltpu.make_async_copy(src_hbm.at[wid, s + 1],
                                  src_v.at[pl.ds(oh, SB)], psem).wait()
            pltpu.make_async_copy(dst_hbm.at[wid, s + 1],
                                  dst_v.at[pl.ds(oh, SB)], psem).wait()
            pltpu.make_async_copy(ex_hbm.at[wid, s + 1],
                                  ex_v.at[pl.ds(oh, SB)], psem).wait()

        @pl.when(ci + 3 < CH)
        def _next():
            nci = ci + 3
            npos = nci % SB
            nhalf = ((nci // SB) % 2) * SB
            pltpu.async_copy(table_hbm.at[src_v.at[nhalf + npos]],
                             rows_v.at[pl.ds(((ci + 3) % 4) * K, K)], gsem)

        rb = b * K
        for g in range(K // 16):
            exvec = ex_v[irow, pl.ds(16 * g, 16)]
            for el in range(16):
                w = jnp.full((16,), exvec[el], _f32)
                for j in range(D // 16):
                    sl = pl.ds(16 * j, 16)
                    r = rb + 16 * g + el
                    rows_v[r, sl] = rows_v[r, sl] * w

        _scatter_fire(irow, b)

    _scatter_drain(SB - 1, (CH - 1) % 4)

    plsc.subcore_barrier()
    pltpu.sync_copy(acc_sh.at[pl.ds(sid * RPS, RPS)],
                    part_hbm.at[cid, pl.ds(sid * RPS, RPS)])


def _elu(v):
    return jnp.where(v > 0, v, jnp.exp(jnp.minimum(v, 0.0)) - 1.0)


def _tc_mid(part_ref, denp_ref, lin2_ref, h2_ref, h3in_ref, dinv_ref):
    dcol = lax.dot_general(denp_ref[...], jnp.ones((NW, 1), _f32),
                           (((0,), (0,)), ((), ())),
                           preferred_element_type=_f32)
    dinv = 1.0 / (dcol + 1e-16)
    dinv_ref[...] = dinv
    agg = part_ref[0, :N, :] + part_ref[1, :N, :]
    h1 = _elu(agg * dinv)
    h2 = jnp.dot(h1, lin2_ref[...], preferred_element_type=_f32)
    h2_ref[...] = h2
    h3in_ref[...] = lax.dot_general(h2, lin2_ref[...],
                                    (((1,), (1,)), ((), ())),
                                    preferred_element_type=_f32)


def _tc_back(part_ref, dinv_ref, lin1_ref, h4_ref):
    agg = part_ref[0, :N, :] + part_ref[1, :N, :]
    h3 = _elu(agg * dinv_ref[...])
    h4_ref[...] = lax.dot_general(h3, lin1_ref[...],
                                  (((1,), (1,)), ((), ())),
                                  preferred_element_type=_f32)


def kernel(x, edge_index, W_in, b_in, lin1, att_src1, att_dst1, lin2):
    src3 = edge_index[0].reshape(NW, NSB, SB, K)
    dst3 = edge_index[1].reshape(NW, NSB, SB, K)

    feats, h, asrc, adst, m = pl.pallas_call(
        _tc_front,
        out_shape=(
            jax.ShapeDtypeStruct((N, D), _f32),
            jax.ShapeDtypeStruct((N, D), _f32),
            jax.ShapeDtypeStruct((1, N), _f32),
            jax.ShapeDtypeStruct((1, N), _f32),
            jax.ShapeDtypeStruct((1, 16), _f32),
        ),
    )(x, W_in, b_in.reshape(1, D), lin1,
      att_src1.reshape(1, D), att_dst1.reshape(1, D))

    ex, denp = _sc_scalar(src3, dst3, asrc, adst, m)

    part1 = _sc_prop(src3, dst3, ex, h)

    h2, h3in, dinv = pl.pallas_call(
        _tc_mid,
        out_shape=(
            jax.ShapeDtypeStruct((N, OUT), _f32),
            jax.ShapeDtypeStruct((N, D), _f32),
            jax.ShapeDtypeStruct((N, 1), _f32),
        ),
    )(part1, denp, lin2)

    part3 = _sc_prop(src3, dst3, ex, h3in)

    h4 = pl.pallas_call(
        _tc_back,
        out_shape=jax.ShapeDtypeStruct((N, D), _f32),
    )(part3, dinv, lin1)

    return (h2, h4, feats)
